# baseline (speedup 1.0000x reference)
import jax
import jax.numpy as jnp
from jax.experimental import pallas as pl

_N = 4194304
_COLS = 1024
_ROWS = _N // _COLS
_BLOCK_ROWS = 128
_GRID = _ROWS // _BLOCK_ROWS


def _mse_body(i_ref, t_ref, o_ref):
    @pl.when(pl.program_id(0) == 0)
    def _init():
        o_ref[...] = jnp.zeros_like(o_ref)

    d = t_ref[...] - i_ref[...]
    o_ref[...] += (jnp.sum(d * d) * (1.0 / _N)).reshape(1, 1)


def kernel(input, target):
    inp2 = input.reshape(_ROWS, _COLS)
    tgt2 = target.reshape(_ROWS, _COLS)
    out = pl.pallas_call(
        _mse_body,
        grid=(_GRID,),
        in_specs=[
            pl.BlockSpec((_BLOCK_ROWS, _COLS), lambda i: (i, 0)),
            pl.BlockSpec((_BLOCK_ROWS, _COLS), lambda i: (i, 0)),
        ],
        out_specs=pl.BlockSpec((1, 1), lambda i: (0, 0)),
        out_shape=jax.ShapeDtypeStruct((1, 1), jnp.float32),
    )(inp2, tgt2)
    return out[0, 0]


# speedup vs baseline: 1.2927x; 1.2927x over previous
import jax
import jax.numpy as jnp
from jax.experimental import pallas as pl

_N = 4194304
_COLS = 1024
_ROWS = _N // _COLS
_BLOCK_ROWS = 1024
_GRID = _ROWS // _BLOCK_ROWS


def _mse_body(i_ref, t_ref, o_ref):
    @pl.when(pl.program_id(0) == 0)
    def _init():
        o_ref[...] = jnp.zeros_like(o_ref)

    d = t_ref[...] - i_ref[...]
    o_ref[...] += (jnp.sum(d * d) * (1.0 / _N)).reshape(1, 1)


def kernel(input, target):
    inp2 = input.reshape(_ROWS, _COLS)
    tgt2 = target.reshape(_ROWS, _COLS)
    out = pl.pallas_call(
        _mse_body,
        grid=(_GRID,),
        in_specs=[
            pl.BlockSpec((_BLOCK_ROWS, _COLS), lambda i: (i, 0)),
            pl.BlockSpec((_BLOCK_ROWS, _COLS), lambda i: (i, 0)),
        ],
        out_specs=pl.BlockSpec((1, 1), lambda i: (0, 0)),
        out_shape=jax.ShapeDtypeStruct((1, 1), jnp.float32),
    )(inp2, tgt2)
    return out[0, 0]


# speedup vs baseline: 1.3028x; 1.0078x over previous
import jax
import jax.numpy as jnp
from jax.experimental import pallas as pl

_N = 4194304
_COLS = 1024
_ROWS = _N // _COLS
_HALF_ROWS = 512
_GRID = _ROWS // (2 * _HALF_ROWS)


def _mse_body(ia_ref, ib_ref, ta_ref, tb_ref, o_ref):
    @pl.when(pl.program_id(0) == 0)
    def _init():
        o_ref[...] = jnp.zeros_like(o_ref)

    da = ta_ref[...] - ia_ref[...]
    db = tb_ref[...] - ib_ref[...]
    s = jnp.sum(da * da) + jnp.sum(db * db)
    o_ref[...] += (s * (1.0 / _N)).reshape(1, 1)


def kernel(input, target):
    inp2 = input.reshape(_ROWS, _COLS)
    tgt2 = target.reshape(_ROWS, _COLS)
    spec_a = pl.BlockSpec((_HALF_ROWS, _COLS), lambda i: (2 * i, 0))
    spec_b = pl.BlockSpec((_HALF_ROWS, _COLS), lambda i: (2 * i + 1, 0))
    out = pl.pallas_call(
        _mse_body,
        grid=(_GRID,),
        in_specs=[spec_a, spec_b, spec_a, spec_b],
        out_specs=pl.BlockSpec((1, 1), lambda i: (0, 0)),
        out_shape=jax.ShapeDtypeStruct((1, 1), jnp.float32),
    )(inp2, inp2, tgt2, tgt2)
    return out[0, 0]
